# baseline (speedup 1.0000x reference)
import functools

import jax
import jax.numpy as jnp
from jax import lax
from jax.experimental import pallas as pl
from jax.experimental.pallas import tpu as pltpu
from jax.experimental.pallas import tpu_sc as plsc
import numpy as np

_Z = np.int32(0)

_N = 10000
_E = 320000
_D = 128
_H = 64
_C = 40
_CP = 48
_K = 10
_ALPHA = 0.1
_NC = 2
_NS = 16
_NW = _NC * _NS
_EPW = _E // _NW
_B = 125
_NCH = _EPW // _B
_NP = 10240
_RPT = _NP // _NS
_BN = 2000


def _sc_mesh():
    return plsc.VectorSubcoreMesh(core_axis_name="c", subcore_axis_name="s",
                                  num_cores=_NC, num_subcores=_NS)


def _deg_body(src_hbm, dst_hbm, ones_hbm, zeros_hbm, osrc_hbm, odst_hbm,
              src_i, dst_i, ones_v, dsrc_sh, ddst_sh):
    c = lax.axis_index("c")
    s = lax.axis_index("s")
    wid = c * _NS + s
    pltpu.sync_copy(src_hbm.at[wid], src_i)
    pltpu.sync_copy(dst_hbm.at[wid], dst_i)
    pltpu.sync_copy(ones_hbm, ones_v)
    rows = pl.ds(s * _RPT, _RPT)
    pltpu.sync_copy(zeros_hbm.at[rows], dsrc_sh.at[rows])
    pltpu.sync_copy(zeros_hbm.at[rows], ddst_sh.at[rows])
    plsc.subcore_barrier()

    def body(j, carry):
        pltpu.sync_copy(ones_v, dsrc_sh.at[src_i.at[j]], add=True)
        pltpu.sync_copy(ones_v, ddst_sh.at[dst_i.at[j]], add=True)
        return carry

    lax.fori_loop(jnp.int32(0), jnp.int32(_NCH), body, jnp.int32(0))
    plsc.subcore_barrier()
    orow = pl.ds(c * _NP + s * _RPT, _RPT)
    pltpu.sync_copy(dsrc_sh.at[rows], osrc_hbm.at[orow])
    pltpu.sync_copy(ddst_sh.at[rows], odst_hbm.at[orow])


def _deg(src, dst, ones16, zeros16):
    f = pl.kernel(
        _deg_body,
        out_type=(jax.ShapeDtypeStruct((_NC * _NP, 16), jnp.float32),
                  jax.ShapeDtypeStruct((_NC * _NP, 16), jnp.float32)),
        mesh=_sc_mesh(),
        compiler_params=pltpu.CompilerParams(use_tc_tiling_on_sc=False),
        scratch_types=[
            pltpu.VMEM((_NCH, _B), jnp.int32),
            pltpu.VMEM((_NCH, _B), jnp.int32),
            pltpu.VMEM((_B, 16), jnp.float32),
            pltpu.VMEM_SHARED((_NP, 16), jnp.float32),
            pltpu.VMEM_SHARED((_NP, 16), jnp.float32),
        ],
    )
    return f(src, dst, ones16, zeros16)


def _step_body(g_hbm, src_hbm, dst_hbm, zeros_hbm, out_hbm,
               src_i, dst_i, buf0, buf1, agg_sh, gs0, gs1, ss0, ss1):
    c = lax.axis_index("c")
    s = lax.axis_index("s")
    wid = c * _NS + s
    pltpu.sync_copy(src_hbm.at[wid], src_i)
    pltpu.sync_copy(dst_hbm.at[wid], dst_i)
    rows = pl.ds(s * _RPT, _RPT)
    pltpu.sync_copy(zeros_hbm.at[rows], agg_sh.at[rows])
    plsc.subcore_barrier()

    pltpu.async_copy(g_hbm.at[src_i.at[jnp.int32(0)]], buf0, gs0)

    def body(jj, carry):
        j = jj * jnp.int32(2)
        pltpu.async_copy(g_hbm.at[src_i.at[j + 1]], buf1, gs1)
        pltpu.make_async_copy(g_hbm.at[src_i.at[j]], buf0, gs0).wait()
        pltpu.sync_copy(buf0, agg_sh.at[dst_i.at[j]], add=True)

        @pl.when(jj + 1 < _NCH // 2)
        def _():
            pltpu.async_copy(g_hbm.at[src_i.at[j + 2]], buf0, gs0)

        pltpu.make_async_copy(g_hbm.at[src_i.at[j + 1]], buf1, gs1).wait()
        pltpu.sync_copy(buf1, agg_sh.at[dst_i.at[j + 1]], add=True)
        return carry

    lax.fori_loop(jnp.int32(0), jnp.int32(_NCH // 2), body, jnp.int32(0))
    plsc.subcore_barrier()
    pltpu.sync_copy(agg_sh.at[rows], out_hbm.at[pl.ds(c * _NP + s * _RPT, _RPT)])


def _step(g, src, dst, zeros):
    f = pl.kernel(
        _step_body,
        out_type=jax.ShapeDtypeStruct((_NC * _NP, _CP), jnp.float32),
        mesh=_sc_mesh(),
        compiler_params=pltpu.CompilerParams(use_tc_tiling_on_sc=False),
        scratch_types=[
            pltpu.VMEM((_NCH, _B), jnp.int32),
            pltpu.VMEM((_NCH, _B), jnp.int32),
            pltpu.VMEM((_B, _CP), jnp.float32),
            pltpu.VMEM((_B, _CP), jnp.float32),
            pltpu.VMEM_SHARED((_NP, _CP), jnp.float32),
            pltpu.SemaphoreType.DMA,
            pltpu.SemaphoreType.DMA,
            pltpu.SemaphoreType.DMA,
            pltpu.SemaphoreType.DMA,
        ],
    )
    return f(g, src, dst, zeros)


_HALF = _NP // _NC
_CRT = _HALF // _NS
_CRH = _CRT // 2


def _xbarrier(xsem, c, s):
    plsc.subcore_barrier()

    @pl.when(s == 0)
    def _():
        pltpu.semaphore_signal(xsem, 1, core_index=jnp.int32(1) - c)
        pltpu.semaphore_wait(xsem, 1)

    plsc.subcore_barrier()


def _fused_body(gin_hbm, bb_hbm, g0t_hbm, af_hbm, c0_hbm, src_hbm, dst_hbm,
                zeros_hbm, g_hbm, x_hbm,
                src_i, dst_i, buf0, buf1, buf2, buf3,
                cb_own, cb_oth, cb_bb, cb_g0t,
                agg_sh, gs0, gs1, gs2, gs3, xsem):
    c = lax.axis_index("c")
    s = lax.axis_index("s")
    wid = c * _NS + s
    rows = pl.ds(s * _RPT, _RPT)
    hrow = c * _HALF + s * _CRT
    crows = pl.ds(hrow, _CRT)
    pltpu.sync_copy(src_hbm.at[wid], src_i)
    pltpu.sync_copy(dst_hbm.at[wid], dst_i)
    pltpu.sync_copy(zeros_hbm.at[rows], agg_sh.at[rows])
    for hh in range(2):
        hc = pl.ds(hrow + hh * _CRH, _CRH)
        pltpu.sync_copy(gin_hbm.at[hc], cb_own)
        pltpu.sync_copy(cb_own, g_hbm.at[hc])
    pltpu.sync_copy(bb_hbm.at[crows], cb_bb)
    pltpu.sync_copy(g0t_hbm.at[crows], cb_g0t)
    _xbarrier(xsem, c, s)

    bufs = (buf0, buf1, buf2, buf3)
    gsems = (gs0, gs1, gs2, gs3)
    _NBUF = 4

    def step(k, carry):
        for t in range(_NBUF - 1):
            pltpu.async_copy(g_hbm.at[src_i.at[jnp.int32(t)]], bufs[t],
                             gsems[t])

        def body(jj, carry2):
            for t in range(_NBUF):
                j = jj * jnp.int32(_NBUF) + t
                nb = (t + _NBUF - 1) % _NBUF
                pltpu.make_async_copy(g_hbm.at[src_i.at[j]], bufs[t],
                                      gsems[t]).wait()

                @pl.when(j + _NBUF - 1 < _NCH)
                def _():
                    pltpu.async_copy(g_hbm.at[src_i.at[j + _NBUF - 1]],
                                     bufs[nb], gsems[nb])

                pltpu.sync_copy(bufs[t], agg_sh.at[dst_i.at[j]], add=True)
            return carry2

        lax.fori_loop(jnp.int32(0), jnp.int32(_NCH // _NBUF), body,
                      jnp.int32(0))
        plsc.subcore_barrier()
        orow = (jnp.int32(1) - c) * _HALF + s * _CRT
        pltpu.sync_copy(agg_sh.at[pl.ds(orow, _CRT)],
                        x_hbm.at[pl.ds(c * _NP + orow, _CRT)])

        @pl.when(k == _K - 1)
        def _():
            pltpu.sync_copy(af_hbm.at[crows], cb_bb)
            pltpu.sync_copy(c0_hbm.at[crows], cb_g0t)

        _xbarrier(xsem, c, s)
        for hh in range(2):
            hc = pl.ds(hrow + hh * _CRH, _CRH)
            pltpu.sync_copy(agg_sh.at[hc], cb_own)
            pltpu.sync_copy(
                x_hbm.at[pl.ds((jnp.int32(1) - c) * _NP + hrow
                               + hh * _CRH, _CRH)], cb_oth)

            def crow(r, carry3, _hh=hh):
                for t in range(_CP // 16):
                    sl = pl.ds(t * 16, 16)
                    rb = r + jnp.int32(_hh * _CRH)
                    cb_own[r, sl] = (cb_bb[rb, sl]
                                     * (cb_own[r, sl] + cb_oth[r, sl])
                                     + cb_g0t[rb, sl])
                return carry3

            lax.fori_loop(jnp.int32(0), jnp.int32(_CRH), crow,
                          jnp.int32(0))
            pltpu.sync_copy(cb_own, g_hbm.at[hc])

        @pl.when(k < _K - 1)
        def _():
            plsc.subcore_barrier()
            pltpu.sync_copy(zeros_hbm.at[rows], agg_sh.at[rows])
            _xbarrier(xsem, c, s)

        return carry

    lax.fori_loop(jnp.int32(0), jnp.int32(_K), step, jnp.int32(0))


def _fused(g, bb, g0t, af, c0, src, dst, zeros):
    f = pl.kernel(
        _fused_body,
        out_type=(jax.ShapeDtypeStruct((_NP, _CP), jnp.float32),
                  jax.ShapeDtypeStruct((_NC * _NP, _CP), jnp.float32)),
        mesh=_sc_mesh(),
        compiler_params=pltpu.CompilerParams(use_tc_tiling_on_sc=False),
        scratch_types=[
            pltpu.VMEM((_NCH, _B), jnp.int32),
            pltpu.VMEM((_NCH, _B), jnp.int32),
            pltpu.VMEM((_B, _CP), jnp.float32),
            pltpu.VMEM((_B, _CP), jnp.float32),
            pltpu.VMEM((_B, _CP), jnp.float32),
            pltpu.VMEM((_B, _CP), jnp.float32),
            pltpu.VMEM((_CRH, _CP), jnp.float32),
            pltpu.VMEM((_CRH, _CP), jnp.float32),
            pltpu.VMEM((_CRT, _CP), jnp.float32),
            pltpu.VMEM((_CRT, _CP), jnp.float32),
            pltpu.VMEM_SHARED((_NP, _CP), jnp.float32),
            pltpu.SemaphoreType.DMA,
            pltpu.SemaphoreType.DMA,
            pltpu.SemaphoreType.DMA,
            pltpu.SemaphoreType.DMA,
            pltpu.SemaphoreType.REGULAR,
        ],
    )
    h, _ = f(g, bb, g0t, af, c0, src, dst, zeros)
    return h


def _mlp_body(x_ref, w0_ref, b0_ref, w1_ref, b1_ref, ps_ref, pd_ref,
              g_ref, g0t_ref, bb_ref, af_ref, c0_ref):
    h = jnp.dot(x_ref[...], w0_ref[...], preferred_element_type=jnp.float32)
    h = jnp.maximum(h + b0_ref[...], 0.0)
    h0 = jnp.dot(h, w1_ref[...], preferred_element_type=jnp.float32) + b1_ref[...]
    degs = ps_ref[0, :, 0] + ps_ref[1, :, 0]
    degd = pd_ref[0, :, 0] + pd_ref[1, :, 0]
    ns = lax.rsqrt(jnp.maximum(degs, 1.0))[:, None]
    nd = lax.rsqrt(jnp.maximum(degd, 1.0))[:, None]
    g_ref[...] = h0 * ns
    g0t_ref[...] = (_ALPHA * ns) * h0
    bb_ref[...] = jnp.broadcast_to((1.0 - _ALPHA) * ns * nd, h0.shape)
    af_ref[...] = jnp.broadcast_to((1.0 - _ALPHA) * nd, h0.shape)
    c0_ref[...] = _ALPHA * h0


def _mlp(x, W0, b0, W1p, b1p, ps, pd):
    grid = _N // _BN
    out = jax.ShapeDtypeStruct((_N, _CP), jnp.float32)
    return pl.pallas_call(
        _mlp_body,
        grid=(grid,),
        in_specs=[
            pl.BlockSpec((_BN, _D), lambda i: (i, _Z)),
            pl.BlockSpec((_D, _H), lambda i: (_Z, _Z)),
            pl.BlockSpec((1, _H), lambda i: (_Z, _Z)),
            pl.BlockSpec((_H, _CP), lambda i: (_Z, _Z)),
            pl.BlockSpec((1, _CP), lambda i: (_Z, _Z)),
            pl.BlockSpec((_NC, _BN, 16), lambda i: (_Z, i, _Z)),
            pl.BlockSpec((_NC, _BN, 16), lambda i: (_Z, i, _Z)),
        ],
        out_specs=[pl.BlockSpec((_BN, _CP), lambda i: (i, _Z))] * 5,
        out_shape=[out] * 5,
    )(x, W0, b0, W1p, b1p, ps, pd)


def _comb_body(p_ref, bb_ref, g0t_ref, g_ref):
    g_ref[...] = bb_ref[...] * (p_ref[0] + p_ref[1]) + g0t_ref[...]


def _combine(p, bb, g0t):
    bc = 1280
    grid = _NP // bc
    return pl.pallas_call(
        _comb_body,
        grid=(grid,),
        in_specs=[
            pl.BlockSpec((_NC, bc, _CP), lambda i: (_Z, i, _Z)),
            pl.BlockSpec((bc, _CP), lambda i: (i, _Z)),
            pl.BlockSpec((bc, _CP), lambda i: (i, _Z)),
        ],
        out_specs=pl.BlockSpec((bc, _CP), lambda i: (i, _Z)),
        out_shape=jax.ShapeDtypeStruct((_NP, _CP), jnp.float32),
    )(p.reshape(_NC, _NP, _CP), bb, g0t)


def _fin_body(p_ref, af_ref, c0_ref, o_ref):
    o_ref[...] = (af_ref[...] * (p_ref[0] + p_ref[1]) + c0_ref[...])[:, :_C]


def _final(p, af, c0):
    grid = _N // _BN
    return pl.pallas_call(
        _fin_body,
        grid=(grid,),
        in_specs=[
            pl.BlockSpec((_NC, _BN, _CP), lambda i: (_Z, i, _Z)),
            pl.BlockSpec((_BN, _CP), lambda i: (i, _Z)),
            pl.BlockSpec((_BN, _CP), lambda i: (i, _Z)),
        ],
        out_specs=pl.BlockSpec((_BN, _C), lambda i: (i, _Z)),
        out_shape=jax.ShapeDtypeStruct((_N, _C), jnp.float32),
    )(p.reshape(_NC, _NP, _CP), af, c0)


def kernel(features, edge_index, W0, b0, W1, b1):
    src = edge_index[0].astype(jnp.int32).reshape(_NW, _NCH, _B)
    dst = edge_index[1].astype(jnp.int32).reshape(_NW, _NCH, _B)
    W1p = jnp.pad(W1, ((0, 0), (0, _CP - _C)))
    b1p = jnp.pad(b1, (0, _CP - _C)).reshape(1, _CP)
    b0r = b0.reshape(1, _H)
    zeros = jnp.zeros((_NP, _CP), jnp.float32)
    zeros16 = jnp.zeros((_NP, 16), jnp.float32)
    ones16 = jnp.ones((_B, 16), jnp.float32)

    psrc, pdst = _deg(src, dst, ones16, zeros16)
    psrc = psrc.reshape(_NC, _NP, 16)
    pdst = pdst.reshape(_NC, _NP, 16)
    g, g0t, bb, af, c0 = _mlp(features.astype(jnp.float32), W0, b0r, W1p, b1p,
                              psrc, pdst)
    padn = ((0, _NP - _N), (0, 0))
    g = jnp.pad(g, padn)
    bb = jnp.pad(bb, padn)
    g0t = jnp.pad(g0t, padn)
    af = jnp.pad(af, padn)
    c0 = jnp.pad(c0, padn)
    h = _fused(g, bb, g0t, af, c0, src, dst, zeros)
    return h[:_N, :_C]


# speedup vs baseline: 1.0039x; 1.0039x over previous
import functools

import jax
import jax.numpy as jnp
from jax import lax
from jax.experimental import pallas as pl
from jax.experimental.pallas import tpu as pltpu
from jax.experimental.pallas import tpu_sc as plsc
import numpy as np

_Z = np.int32(0)

_N = 10000
_E = 320000
_D = 128
_H = 64
_C = 40
_CP = 48
_K = 10
_ALPHA = 0.1
_NC = 2
_NS = 16
_NW = _NC * _NS
_EPW = _E // _NW
_B = 125
_NCH = _EPW // _B
_NP = 10240
_RPT = _NP // _NS
_BN = 2000


def _sc_mesh():
    return plsc.VectorSubcoreMesh(core_axis_name="c", subcore_axis_name="s",
                                  num_cores=_NC, num_subcores=_NS)


def _deg_body(src_hbm, dst_hbm, ones_hbm, zeros_hbm, osrc_hbm, odst_hbm,
              src_i, dst_i, ones_v, dsrc_sh, ddst_sh):
    c = lax.axis_index("c")
    s = lax.axis_index("s")
    wid = c * _NS + s
    pltpu.sync_copy(src_hbm.at[wid], src_i)
    pltpu.sync_copy(dst_hbm.at[wid], dst_i)
    pltpu.sync_copy(ones_hbm, ones_v)
    rows = pl.ds(s * _RPT, _RPT)
    pltpu.sync_copy(zeros_hbm.at[rows], dsrc_sh.at[rows])
    pltpu.sync_copy(zeros_hbm.at[rows], ddst_sh.at[rows])
    plsc.subcore_barrier()

    def body(j, carry):
        pltpu.sync_copy(ones_v, dsrc_sh.at[src_i.at[j]], add=True)
        pltpu.sync_copy(ones_v, ddst_sh.at[dst_i.at[j]], add=True)
        return carry

    lax.fori_loop(jnp.int32(0), jnp.int32(_NCH), body, jnp.int32(0))
    plsc.subcore_barrier()
    orow = pl.ds(c * _NP + s * _RPT, _RPT)
    pltpu.sync_copy(dsrc_sh.at[rows], osrc_hbm.at[orow])
    pltpu.sync_copy(ddst_sh.at[rows], odst_hbm.at[orow])


def _deg(src, dst, ones16, zeros16):
    f = pl.kernel(
        _deg_body,
        out_type=(jax.ShapeDtypeStruct((_NC * _NP, 16), jnp.float32),
                  jax.ShapeDtypeStruct((_NC * _NP, 16), jnp.float32)),
        mesh=_sc_mesh(),
        compiler_params=pltpu.CompilerParams(use_tc_tiling_on_sc=False),
        scratch_types=[
            pltpu.VMEM((_NCH, _B), jnp.int32),
            pltpu.VMEM((_NCH, _B), jnp.int32),
            pltpu.VMEM((_B, 16), jnp.float32),
            pltpu.VMEM_SHARED((_NP, 16), jnp.float32),
            pltpu.VMEM_SHARED((_NP, 16), jnp.float32),
        ],
    )
    return f(src, dst, ones16, zeros16)


def _step_body(g_hbm, src_hbm, dst_hbm, zeros_hbm, out_hbm,
               src_i, dst_i, buf0, buf1, agg_sh, gs0, gs1, ss0, ss1):
    c = lax.axis_index("c")
    s = lax.axis_index("s")
    wid = c * _NS + s
    pltpu.sync_copy(src_hbm.at[wid], src_i)
    pltpu.sync_copy(dst_hbm.at[wid], dst_i)
    rows = pl.ds(s * _RPT, _RPT)
    pltpu.sync_copy(zeros_hbm.at[rows], agg_sh.at[rows])
    plsc.subcore_barrier()

    pltpu.async_copy(g_hbm.at[src_i.at[jnp.int32(0)]], buf0, gs0)

    def body(jj, carry):
        j = jj * jnp.int32(2)
        pltpu.async_copy(g_hbm.at[src_i.at[j + 1]], buf1, gs1)
        pltpu.make_async_copy(g_hbm.at[src_i.at[j]], buf0, gs0).wait()
        pltpu.sync_copy(buf0, agg_sh.at[dst_i.at[j]], add=True)

        @pl.when(jj + 1 < _NCH // 2)
        def _():
            pltpu.async_copy(g_hbm.at[src_i.at[j + 2]], buf0, gs0)

        pltpu.make_async_copy(g_hbm.at[src_i.at[j + 1]], buf1, gs1).wait()
        pltpu.sync_copy(buf1, agg_sh.at[dst_i.at[j + 1]], add=True)
        return carry

    lax.fori_loop(jnp.int32(0), jnp.int32(_NCH // 2), body, jnp.int32(0))
    plsc.subcore_barrier()
    pltpu.sync_copy(agg_sh.at[rows], out_hbm.at[pl.ds(c * _NP + s * _RPT, _RPT)])


def _step(g, src, dst, zeros):
    f = pl.kernel(
        _step_body,
        out_type=jax.ShapeDtypeStruct((_NC * _NP, _CP), jnp.float32),
        mesh=_sc_mesh(),
        compiler_params=pltpu.CompilerParams(use_tc_tiling_on_sc=False),
        scratch_types=[
            pltpu.VMEM((_NCH, _B), jnp.int32),
            pltpu.VMEM((_NCH, _B), jnp.int32),
            pltpu.VMEM((_B, _CP), jnp.float32),
            pltpu.VMEM((_B, _CP), jnp.float32),
            pltpu.VMEM_SHARED((_NP, _CP), jnp.float32),
            pltpu.SemaphoreType.DMA,
            pltpu.SemaphoreType.DMA,
            pltpu.SemaphoreType.DMA,
            pltpu.SemaphoreType.DMA,
        ],
    )
    return f(g, src, dst, zeros)


_HALF = _NP // _NC
_CRT = _HALF // _NS
_CRH = _CRT // 2


def _xbarrier(xsem, c, s):
    plsc.subcore_barrier()

    @pl.when(s == 0)
    def _():
        pltpu.semaphore_signal(xsem, 1, core_index=jnp.int32(1) - c)
        pltpu.semaphore_wait(xsem, 1)

    plsc.subcore_barrier()


def _fused_body(gin_hbm, bb_hbm, g0t_hbm, af_hbm, c0_hbm, src_hbm, dst_hbm,
                zeros_hbm, g_hbm, x_hbm,
                src_i, dst_i, buf0, buf1, buf2, buf3,
                cb_own, cb_oth, cb_own1, cb_oth1, cb_bb, cb_g0t,
                agg_sh, gs0, gs1, gs2, gs3, xsem):
    c = lax.axis_index("c")
    s = lax.axis_index("s")
    wid = c * _NS + s
    rows = pl.ds(s * _RPT, _RPT)
    hrow = c * _HALF + s * _CRT
    crows = pl.ds(hrow, _CRT)
    pltpu.sync_copy(src_hbm.at[wid], src_i)
    pltpu.sync_copy(dst_hbm.at[wid], dst_i)
    pltpu.sync_copy(zeros_hbm.at[rows], agg_sh.at[rows])
    for hh in range(2):
        hc = pl.ds(hrow + hh * _CRH, _CRH)
        pltpu.sync_copy(gin_hbm.at[hc], cb_own)
        pltpu.sync_copy(cb_own, g_hbm.at[hc])
    pltpu.sync_copy(bb_hbm.at[crows], cb_bb)
    _xbarrier(xsem, c, s)

    bufs = (buf0, buf1, buf2, buf3)
    gsems = (gs0, gs1, gs2, gs3)
    _NBUF = 4

    def step(k, carry):
        for t in range(_NBUF - 1):
            pltpu.async_copy(g_hbm.at[src_i.at[jnp.int32(t)]], bufs[t],
                             gsems[t])

        def body(jj, carry2):
            for t in range(_NBUF):
                j = jj * jnp.int32(_NBUF) + t
                nb = (t + _NBUF - 1) % _NBUF
                pltpu.make_async_copy(g_hbm.at[src_i.at[j]], bufs[t],
                                      gsems[t]).wait()

                @pl.when(j + _NBUF - 1 < _NCH)
                def _():
                    pltpu.async_copy(g_hbm.at[src_i.at[j + _NBUF - 1]],
                                     bufs[nb], gsems[nb])

                pltpu.sync_copy(bufs[t], agg_sh.at[dst_i.at[j]], add=True)
            return carry2

        lax.fori_loop(jnp.int32(0), jnp.int32(_NCH // _NBUF), body,
                      jnp.int32(0))
        plsc.subcore_barrier()
        orow = (jnp.int32(1) - c) * _HALF + s * _CRT
        pltpu.sync_copy(agg_sh.at[pl.ds(orow, _CRT)],
                        x_hbm.at[pl.ds(c * _NP + orow, _CRT)])

        @pl.when(k == _K - 1)
        def _():
            pltpu.sync_copy(af_hbm.at[crows], cb_bb)

        _xbarrier(xsem, c, s)
        oth_base = (jnp.int32(1) - c) * _NP + hrow
        owns = (cb_own, cb_own1)
        oths = (cb_oth, cb_oth1)
        for hh in range(2):
            hc = pl.ds(hrow + hh * _CRH, _CRH)
            pltpu.async_copy(agg_sh.at[hc], owns[hh], gsems[2 * hh])
            pltpu.async_copy(x_hbm.at[pl.ds(oth_base + hh * _CRH, _CRH)],
                             oths[hh], gsems[2 * hh + 1])

        @pl.when(k < _K - 1)
        def _():
            pltpu.sync_copy(zeros_hbm.at[pl.ds(orow, _CRT)],
                            agg_sh.at[pl.ds(orow, _CRT)])

        for hh in range(2):
            hc = pl.ds(hrow + hh * _CRH, _CRH)
            pltpu.make_async_copy(agg_sh.at[hc], owns[hh],
                                  gsems[2 * hh]).wait()

            @pl.when(k < _K - 1)
            def _(_hh=hh, _hc=hc):
                pltpu.sync_copy(zeros_hbm.at[_hc], agg_sh.at[_hc])

            hcg = pl.ds(hrow + hh * _CRH, _CRH)

            @pl.when(k < _K - 1)
            def _(_hcg=hcg):
                pltpu.sync_copy(g0t_hbm.at[_hcg], cb_g0t)

            @pl.when(k == _K - 1)
            def _(_hcg=hcg):
                pltpu.sync_copy(c0_hbm.at[_hcg], cb_g0t)

            pltpu.make_async_copy(
                x_hbm.at[pl.ds(oth_base + hh * _CRH, _CRH)], oths[hh],
                gsems[2 * hh + 1]).wait()

            def crow(r, carry3, _hh=hh):
                for t in range(_CP // 16):
                    sl = pl.ds(t * 16, 16)
                    rb = r + jnp.int32(_hh * _CRH)
                    owns[_hh][r, sl] = (cb_bb[rb, sl]
                                        * (owns[_hh][r, sl]
                                           + oths[_hh][r, sl])
                                        + cb_g0t[r, sl])
                return carry3

            lax.fori_loop(jnp.int32(0), jnp.int32(_CRH), crow,
                          jnp.int32(0))
            pltpu.sync_copy(owns[hh], g_hbm.at[hc])

        @pl.when(k < _K - 1)
        def _():
            _xbarrier(xsem, c, s)

        return carry

    lax.fori_loop(jnp.int32(0), jnp.int32(_K), step, jnp.int32(0))


def _fused(g, bb, g0t, af, c0, src, dst, zeros):
    f = pl.kernel(
        _fused_body,
        out_type=(jax.ShapeDtypeStruct((_NP, _CP), jnp.float32),
                  jax.ShapeDtypeStruct((_NC * _NP, _CP), jnp.float32)),
        mesh=_sc_mesh(),
        compiler_params=pltpu.CompilerParams(use_tc_tiling_on_sc=False),
        scratch_types=[
            pltpu.VMEM((_NCH, _B), jnp.int32),
            pltpu.VMEM((_NCH, _B), jnp.int32),
            pltpu.VMEM((_B, _CP), jnp.float32),
            pltpu.VMEM((_B, _CP), jnp.float32),
            pltpu.VMEM((_B, _CP), jnp.float32),
            pltpu.VMEM((_B, _CP), jnp.float32),
            pltpu.VMEM((_CRH, _CP), jnp.float32),
            pltpu.VMEM((_CRH, _CP), jnp.float32),
            pltpu.VMEM((_CRH, _CP), jnp.float32),
            pltpu.VMEM((_CRH, _CP), jnp.float32),
            pltpu.VMEM((_CRT, _CP), jnp.float32),
            pltpu.VMEM((_CRH, _CP), jnp.float32),
            pltpu.VMEM_SHARED((_NP, _CP), jnp.float32),
            pltpu.SemaphoreType.DMA,
            pltpu.SemaphoreType.DMA,
            pltpu.SemaphoreType.DMA,
            pltpu.SemaphoreType.DMA,
            pltpu.SemaphoreType.REGULAR,
        ],
    )
    h, _ = f(g, bb, g0t, af, c0, src, dst, zeros)
    return h


def _mlp_body(x_ref, w0_ref, b0_ref, w1_ref, b1_ref, ps_ref, pd_ref,
              g_ref, g0t_ref, bb_ref, af_ref, c0_ref):
    h = jnp.dot(x_ref[...], w0_ref[...], preferred_element_type=jnp.float32)
    h = jnp.maximum(h + b0_ref[...], 0.0)
    h0 = jnp.dot(h, w1_ref[...], preferred_element_type=jnp.float32) + b1_ref[...]
    degs = ps_ref[0, :, 0] + ps_ref[1, :, 0]
    degd = pd_ref[0, :, 0] + pd_ref[1, :, 0]
    ns = lax.rsqrt(jnp.maximum(degs, 1.0))[:, None]
    nd = lax.rsqrt(jnp.maximum(degd, 1.0))[:, None]
    g_ref[...] = h0 * ns
    g0t_ref[...] = (_ALPHA * ns) * h0
    bb_ref[...] = jnp.broadcast_to((1.0 - _ALPHA) * ns * nd, h0.shape)
    af_ref[...] = jnp.broadcast_to((1.0 - _ALPHA) * nd, h0.shape)
    c0_ref[...] = _ALPHA * h0


def _mlp(x, W0, b0, W1p, b1p, ps, pd):
    grid = _N // _BN
    out = jax.ShapeDtypeStruct((_N, _CP), jnp.float32)
    return pl.pallas_call(
        _mlp_body,
        grid=(grid,),
        in_specs=[
            pl.BlockSpec((_BN, _D), lambda i: (i, _Z)),
            pl.BlockSpec((_D, _H), lambda i: (_Z, _Z)),
            pl.BlockSpec((1, _H), lambda i: (_Z, _Z)),
            pl.BlockSpec((_H, _CP), lambda i: (_Z, _Z)),
            pl.BlockSpec((1, _CP), lambda i: (_Z, _Z)),
            pl.BlockSpec((_NC, _BN, 16), lambda i: (_Z, i, _Z)),
            pl.BlockSpec((_NC, _BN, 16), lambda i: (_Z, i, _Z)),
        ],
        out_specs=[pl.BlockSpec((_BN, _CP), lambda i: (i, _Z))] * 5,
        out_shape=[out] * 5,
    )(x, W0, b0, W1p, b1p, ps, pd)


def _comb_body(p_ref, bb_ref, g0t_ref, g_ref):
    g_ref[...] = bb_ref[...] * (p_ref[0] + p_ref[1]) + g0t_ref[...]


def _combine(p, bb, g0t):
    bc = 1280
    grid = _NP // bc
    return pl.pallas_call(
        _comb_body,
        grid=(grid,),
        in_specs=[
            pl.BlockSpec((_NC, bc, _CP), lambda i: (_Z, i, _Z)),
            pl.BlockSpec((bc, _CP), lambda i: (i, _Z)),
            pl.BlockSpec((bc, _CP), lambda i: (i, _Z)),
        ],
        out_specs=pl.BlockSpec((bc, _CP), lambda i: (i, _Z)),
        out_shape=jax.ShapeDtypeStruct((_NP, _CP), jnp.float32),
    )(p.reshape(_NC, _NP, _CP), bb, g0t)


def _fin_body(p_ref, af_ref, c0_ref, o_ref):
    o_ref[...] = (af_ref[...] * (p_ref[0] + p_ref[1]) + c0_ref[...])[:, :_C]


def _final(p, af, c0):
    grid = _N // _BN
    return pl.pallas_call(
        _fin_body,
        grid=(grid,),
        in_specs=[
            pl.BlockSpec((_NC, _BN, _CP), lambda i: (_Z, i, _Z)),
            pl.BlockSpec((_BN, _CP), lambda i: (i, _Z)),
            pl.BlockSpec((_BN, _CP), lambda i: (i, _Z)),
        ],
        out_specs=pl.BlockSpec((_BN, _C), lambda i: (i, _Z)),
        out_shape=jax.ShapeDtypeStruct((_N, _C), jnp.float32),
    )(p.reshape(_NC, _NP, _CP), af, c0)


def kernel(features, edge_index, W0, b0, W1, b1):
    src = edge_index[0].astype(jnp.int32).reshape(_NW, _NCH, _B)
    dst = edge_index[1].astype(jnp.int32).reshape(_NW, _NCH, _B)
    W1p = jnp.pad(W1, ((0, 0), (0, _CP - _C)))
    b1p = jnp.pad(b1, (0, _CP - _C)).reshape(1, _CP)
    b0r = b0.reshape(1, _H)
    zeros = jnp.zeros((_NP, _CP), jnp.float32)
    zeros16 = jnp.zeros((_NP, 16), jnp.float32)
    ones16 = jnp.ones((_B, 16), jnp.float32)

    psrc, pdst = _deg(src, dst, ones16, zeros16)
    psrc = psrc.reshape(_NC, _NP, 16)
    pdst = pdst.reshape(_NC, _NP, 16)
    g, g0t, bb, af, c0 = _mlp(features.astype(jnp.float32), W0, b0r, W1p, b1p,
                              psrc, pdst)
    padn = ((0, _NP - _N), (0, 0))
    g = jnp.pad(g, padn)
    bb = jnp.pad(bb, padn)
    g0t = jnp.pad(g0t, padn)
    af = jnp.pad(af, padn)
    c0 = jnp.pad(c0, padn)
    h = _fused(g, bb, g0t, af, c0, src, dst, zeros)
    return h[:_N, :_C]


# speedup vs baseline: 1.0210x; 1.0170x over previous
import functools

import jax
import jax.numpy as jnp
from jax import lax
from jax.experimental import pallas as pl
from jax.experimental.pallas import tpu as pltpu
from jax.experimental.pallas import tpu_sc as plsc
import numpy as np

_Z = np.int32(0)

_N = 10000
_E = 320000
_D = 128
_H = 64
_C = 40
_CP = 48
_K = 10
_ALPHA = 0.1
_NC = 2
_NS = 16
_NW = _NC * _NS
_EPW = _E // _NW
_B = 125
_NCH = _EPW // _B
_NP = 10240
_RPT = _NP // _NS
_BN = 2000


def _sc_mesh():
    return plsc.VectorSubcoreMesh(core_axis_name="c", subcore_axis_name="s",
                                  num_cores=_NC, num_subcores=_NS)


def _deg_body(src_hbm, dst_hbm, ones_hbm, zeros_hbm, osrc_hbm, odst_hbm,
              src_i, dst_i, ones_v, dsrc_sh, ddst_sh):
    c = lax.axis_index("c")
    s = lax.axis_index("s")
    wid = c * _NS + s
    pltpu.sync_copy(src_hbm.at[wid], src_i)
    pltpu.sync_copy(dst_hbm.at[wid], dst_i)
    pltpu.sync_copy(ones_hbm, ones_v)
    rows = pl.ds(s * _RPT, _RPT)
    pltpu.sync_copy(zeros_hbm.at[rows], dsrc_sh.at[rows])
    pltpu.sync_copy(zeros_hbm.at[rows], ddst_sh.at[rows])
    plsc.subcore_barrier()

    def body(j, carry):
        pltpu.sync_copy(ones_v, dsrc_sh.at[src_i.at[j]], add=True)
        pltpu.sync_copy(ones_v, ddst_sh.at[dst_i.at[j]], add=True)
        return carry

    lax.fori_loop(jnp.int32(0), jnp.int32(_NCH), body, jnp.int32(0))
    plsc.subcore_barrier()
    orow = pl.ds(c * _NP + s * _RPT, _RPT)
    pltpu.sync_copy(dsrc_sh.at[rows], osrc_hbm.at[orow])
    pltpu.sync_copy(ddst_sh.at[rows], odst_hbm.at[orow])


def _deg(src, dst, ones16, zeros16):
    f = pl.kernel(
        _deg_body,
        out_type=(jax.ShapeDtypeStruct((_NC * _NP, 16), jnp.float32),
                  jax.ShapeDtypeStruct((_NC * _NP, 16), jnp.float32)),
        mesh=_sc_mesh(),
        compiler_params=pltpu.CompilerParams(use_tc_tiling_on_sc=False),
        scratch_types=[
            pltpu.VMEM((_NCH, _B), jnp.int32),
            pltpu.VMEM((_NCH, _B), jnp.int32),
            pltpu.VMEM((_B, 16), jnp.float32),
            pltpu.VMEM_SHARED((_NP, 16), jnp.float32),
            pltpu.VMEM_SHARED((_NP, 16), jnp.float32),
        ],
    )
    return f(src, dst, ones16, zeros16)


def _step_body(g_hbm, src_hbm, dst_hbm, zeros_hbm, out_hbm,
               src_i, dst_i, buf0, buf1, agg_sh, gs0, gs1, ss0, ss1):
    c = lax.axis_index("c")
    s = lax.axis_index("s")
    wid = c * _NS + s
    pltpu.sync_copy(src_hbm.at[wid], src_i)
    pltpu.sync_copy(dst_hbm.at[wid], dst_i)
    rows = pl.ds(s * _RPT, _RPT)
    pltpu.sync_copy(zeros_hbm.at[rows], agg_sh.at[rows])
    plsc.subcore_barrier()

    pltpu.async_copy(g_hbm.at[src_i.at[jnp.int32(0)]], buf0, gs0)

    def body(jj, carry):
        j = jj * jnp.int32(2)
        pltpu.async_copy(g_hbm.at[src_i.at[j + 1]], buf1, gs1)
        pltpu.make_async_copy(g_hbm.at[src_i.at[j]], buf0, gs0).wait()
        pltpu.sync_copy(buf0, agg_sh.at[dst_i.at[j]], add=True)

        @pl.when(jj + 1 < _NCH // 2)
        def _():
            pltpu.async_copy(g_hbm.at[src_i.at[j + 2]], buf0, gs0)

        pltpu.make_async_copy(g_hbm.at[src_i.at[j + 1]], buf1, gs1).wait()
        pltpu.sync_copy(buf1, agg_sh.at[dst_i.at[j + 1]], add=True)
        return carry

    lax.fori_loop(jnp.int32(0), jnp.int32(_NCH // 2), body, jnp.int32(0))
    plsc.subcore_barrier()
    pltpu.sync_copy(agg_sh.at[rows], out_hbm.at[pl.ds(c * _NP + s * _RPT, _RPT)])


def _step(g, src, dst, zeros):
    f = pl.kernel(
        _step_body,
        out_type=jax.ShapeDtypeStruct((_NC * _NP, _CP), jnp.float32),
        mesh=_sc_mesh(),
        compiler_params=pltpu.CompilerParams(use_tc_tiling_on_sc=False),
        scratch_types=[
            pltpu.VMEM((_NCH, _B), jnp.int32),
            pltpu.VMEM((_NCH, _B), jnp.int32),
            pltpu.VMEM((_B, _CP), jnp.float32),
            pltpu.VMEM((_B, _CP), jnp.float32),
            pltpu.VMEM_SHARED((_NP, _CP), jnp.float32),
            pltpu.SemaphoreType.DMA,
            pltpu.SemaphoreType.DMA,
            pltpu.SemaphoreType.DMA,
            pltpu.SemaphoreType.DMA,
        ],
    )
    return f(g, src, dst, zeros)


_HALF = _NP // _NC
_CRT = _HALF // _NS
_CRH = _CRT // 2


def _xbarrier(xsem, c, s):
    plsc.subcore_barrier()

    @pl.when(s == 0)
    def _():
        pltpu.semaphore_signal(xsem, 1, core_index=jnp.int32(1) - c)
        pltpu.semaphore_wait(xsem, 1)

    plsc.subcore_barrier()


def _fused_body(gin_hbm, bb_hbm, g0t_hbm, src_hbm, dst_hbm, zeros_hbm,
                g_hbm, x_hbm,
                src_i, dst_i, buf0, buf1, buf2, buf3,
                cb_own, cb_oth, cb_own1, cb_oth1, cb_bb, cb_g0t,
                agg_sh, gs0, gs1, gs2, gs3, xsem):
    c = lax.axis_index("c")
    s = lax.axis_index("s")
    wid = c * _NS + s
    rows = pl.ds(s * _RPT, _RPT)
    hrow = c * _HALF + s * _CRT
    crows = pl.ds(hrow, _CRT)
    pltpu.sync_copy(src_hbm.at[wid], src_i)
    pltpu.sync_copy(dst_hbm.at[wid], dst_i)
    pltpu.sync_copy(zeros_hbm.at[rows], agg_sh.at[rows])
    for hh in range(2):
        hc = pl.ds(hrow + hh * _CRH, _CRH)
        pltpu.sync_copy(gin_hbm.at[hc], cb_own)
        pltpu.sync_copy(cb_own, g_hbm.at[hc])
    pltpu.sync_copy(bb_hbm.at[crows], cb_bb)
    _xbarrier(xsem, c, s)

    bufs = (buf0, buf1, buf2, buf3)
    gsems = (gs0, gs1, gs2, gs3)
    _NBUF = 4

    def step(k, carry):
        for t in range(_NBUF - 1):
            pltpu.async_copy(g_hbm.at[src_i.at[jnp.int32(t)]], bufs[t],
                             gsems[t])

        def body(jj, carry2):
            for t in range(_NBUF):
                j = jj * jnp.int32(_NBUF) + t
                nb = (t + _NBUF - 1) % _NBUF
                pltpu.make_async_copy(g_hbm.at[src_i.at[j]], bufs[t],
                                      gsems[t]).wait()

                @pl.when(j + _NBUF - 1 < _NCH)
                def _():
                    pltpu.async_copy(g_hbm.at[src_i.at[j + _NBUF - 1]],
                                     bufs[nb], gsems[nb])

                pltpu.sync_copy(bufs[t], agg_sh.at[dst_i.at[j]], add=True)
            return carry2

        lax.fori_loop(jnp.int32(0), jnp.int32(_NCH // _NBUF), body,
                      jnp.int32(0))
        plsc.subcore_barrier()
        orow = (jnp.int32(1) - c) * _HALF + s * _CRT
        pltpu.sync_copy(agg_sh.at[pl.ds(orow, _CRT)],
                        x_hbm.at[pl.ds(c * _NP + orow, _CRT)])

        @pl.when(k == _K - 1)
        def _():
            pltpu.sync_copy(agg_sh.at[crows],
                            x_hbm.at[pl.ds(c * _NP + hrow, _CRT)])

        @pl.when(k < _K - 1)
        def _():
            _xbarrier(xsem, c, s)
            oth_base = (jnp.int32(1) - c) * _NP + hrow
            owns = (cb_own, cb_own1)
            oths = (cb_oth, cb_oth1)
            for hh in range(2):
                hc = pl.ds(hrow + hh * _CRH, _CRH)
                pltpu.async_copy(agg_sh.at[hc], owns[hh], gsems[2 * hh])
                pltpu.async_copy(
                    x_hbm.at[pl.ds(oth_base + hh * _CRH, _CRH)], oths[hh],
                    gsems[2 * hh + 1])
            pltpu.sync_copy(zeros_hbm.at[pl.ds(orow, _CRT)],
                            agg_sh.at[pl.ds(orow, _CRT)])
            for hh in range(2):
                hc = pl.ds(hrow + hh * _CRH, _CRH)
                pltpu.make_async_copy(agg_sh.at[hc], owns[hh],
                                      gsems[2 * hh]).wait()
                pltpu.sync_copy(zeros_hbm.at[hc], agg_sh.at[hc])
                pltpu.sync_copy(g0t_hbm.at[hc], cb_g0t)
                pltpu.make_async_copy(
                    x_hbm.at[pl.ds(oth_base + hh * _CRH, _CRH)], oths[hh],
                    gsems[2 * hh + 1]).wait()

                def crow(r, carry3, _hh=hh):
                    for t in range(_CP // 16):
                        sl = pl.ds(t * 16, 16)
                        rb = r + jnp.int32(_hh * _CRH)
                        owns[_hh][r, sl] = (cb_bb[rb, sl]
                                            * (owns[_hh][r, sl]
                                               + oths[_hh][r, sl])
                                            + cb_g0t[r, sl])
                    return carry3

                lax.fori_loop(jnp.int32(0), jnp.int32(_CRH), crow,
                              jnp.int32(0))
                pltpu.sync_copy(owns[hh], g_hbm.at[hc])
            _xbarrier(xsem, c, s)

        return carry

    lax.fori_loop(jnp.int32(0), jnp.int32(_K), step, jnp.int32(0))


def _fused(g, bb, g0t, src, dst, zeros):
    f = pl.kernel(
        _fused_body,
        out_type=(jax.ShapeDtypeStruct((_NP, _CP), jnp.float32),
                  jax.ShapeDtypeStruct((_NC * _NP, _CP), jnp.float32)),
        mesh=_sc_mesh(),
        compiler_params=pltpu.CompilerParams(use_tc_tiling_on_sc=False),
        scratch_types=[
            pltpu.VMEM((_NCH, _B), jnp.int32),
            pltpu.VMEM((_NCH, _B), jnp.int32),
            pltpu.VMEM((_B, _CP), jnp.float32),
            pltpu.VMEM((_B, _CP), jnp.float32),
            pltpu.VMEM((_B, _CP), jnp.float32),
            pltpu.VMEM((_B, _CP), jnp.float32),
            pltpu.VMEM((_CRH, _CP), jnp.float32),
            pltpu.VMEM((_CRH, _CP), jnp.float32),
            pltpu.VMEM((_CRH, _CP), jnp.float32),
            pltpu.VMEM((_CRH, _CP), jnp.float32),
            pltpu.VMEM((_CRT, _CP), jnp.float32),
            pltpu.VMEM((_CRH, _CP), jnp.float32),
            pltpu.VMEM_SHARED((_NP, _CP), jnp.float32),
            pltpu.SemaphoreType.DMA,
            pltpu.SemaphoreType.DMA,
            pltpu.SemaphoreType.DMA,
            pltpu.SemaphoreType.DMA,
            pltpu.SemaphoreType.REGULAR,
        ],
    )
    _, x = f(g, bb, g0t, src, dst, zeros)
    return x


def _mlp_body(x_ref, w0_ref, b0_ref, w1_ref, b1_ref, ps_ref, pd_ref,
              g_ref, g0t_ref, bb_ref, af_ref, c0_ref):
    h = jnp.dot(x_ref[...], w0_ref[...], preferred_element_type=jnp.float32)
    h = jnp.maximum(h + b0_ref[...], 0.0)
    h0 = jnp.dot(h, w1_ref[...], preferred_element_type=jnp.float32) + b1_ref[...]
    degs = ps_ref[0, :, 0] + ps_ref[1, :, 0]
    degd = pd_ref[0, :, 0] + pd_ref[1, :, 0]
    ns = lax.rsqrt(jnp.maximum(degs, 1.0))[:, None]
    nd = lax.rsqrt(jnp.maximum(degd, 1.0))[:, None]
    g_ref[...] = h0 * ns
    g0t_ref[...] = (_ALPHA * ns) * h0
    bb_ref[...] = jnp.broadcast_to((1.0 - _ALPHA) * ns * nd, h0.shape)
    af_ref[...] = jnp.broadcast_to((1.0 - _ALPHA) * nd, h0.shape)
    c0_ref[...] = _ALPHA * h0


def _mlp(x, W0, b0, W1p, b1p, ps, pd):
    grid = _N // _BN
    out = jax.ShapeDtypeStruct((_N, _CP), jnp.float32)
    return pl.pallas_call(
        _mlp_body,
        grid=(grid,),
        in_specs=[
            pl.BlockSpec((_BN, _D), lambda i: (i, _Z)),
            pl.BlockSpec((_D, _H), lambda i: (_Z, _Z)),
            pl.BlockSpec((1, _H), lambda i: (_Z, _Z)),
            pl.BlockSpec((_H, _CP), lambda i: (_Z, _Z)),
            pl.BlockSpec((1, _CP), lambda i: (_Z, _Z)),
            pl.BlockSpec((_NC, _BN, 16), lambda i: (_Z, i, _Z)),
            pl.BlockSpec((_NC, _BN, 16), lambda i: (_Z, i, _Z)),
        ],
        out_specs=[pl.BlockSpec((_BN, _CP), lambda i: (i, _Z))] * 5,
        out_shape=[out] * 5,
    )(x, W0, b0, W1p, b1p, ps, pd)


def _comb_body(p_ref, bb_ref, g0t_ref, g_ref):
    g_ref[...] = bb_ref[...] * (p_ref[0] + p_ref[1]) + g0t_ref[...]


def _combine(p, bb, g0t):
    bc = 1280
    grid = _NP // bc
    return pl.pallas_call(
        _comb_body,
        grid=(grid,),
        in_specs=[
            pl.BlockSpec((_NC, bc, _CP), lambda i: (_Z, i, _Z)),
            pl.BlockSpec((bc, _CP), lambda i: (i, _Z)),
            pl.BlockSpec((bc, _CP), lambda i: (i, _Z)),
        ],
        out_specs=pl.BlockSpec((bc, _CP), lambda i: (i, _Z)),
        out_shape=jax.ShapeDtypeStruct((_NP, _CP), jnp.float32),
    )(p.reshape(_NC, _NP, _CP), bb, g0t)


def _fin_body(p_ref, af_ref, c0_ref, o_ref):
    o_ref[...] = (af_ref[...] * (p_ref[0] + p_ref[1]) + c0_ref[...])[:, :_C]


def _final(p, af, c0):
    grid = _N // _BN
    return pl.pallas_call(
        _fin_body,
        grid=(grid,),
        in_specs=[
            pl.BlockSpec((_NC, _BN, _CP), lambda i: (_Z, i, _Z)),
            pl.BlockSpec((_BN, _CP), lambda i: (i, _Z)),
            pl.BlockSpec((_BN, _CP), lambda i: (i, _Z)),
        ],
        out_specs=pl.BlockSpec((_BN, _C), lambda i: (i, _Z)),
        out_shape=jax.ShapeDtypeStruct((_N, _C), jnp.float32),
    )(p.reshape(_NC, _NP, _CP), af, c0)


def kernel(features, edge_index, W0, b0, W1, b1):
    src = edge_index[0].astype(jnp.int32).reshape(_NW, _NCH, _B)
    dst = edge_index[1].astype(jnp.int32).reshape(_NW, _NCH, _B)
    W1p = jnp.pad(W1, ((0, 0), (0, _CP - _C)))
    b1p = jnp.pad(b1, (0, _CP - _C)).reshape(1, _CP)
    b0r = b0.reshape(1, _H)
    zeros = jnp.zeros((_NP, _CP), jnp.float32)
    zeros16 = jnp.zeros((_NP, 16), jnp.float32)
    ones16 = jnp.ones((_B, 16), jnp.float32)

    psrc, pdst = _deg(src, dst, ones16, zeros16)
    psrc = psrc.reshape(_NC, _NP, 16)
    pdst = pdst.reshape(_NC, _NP, 16)
    g, g0t, bb, af, c0 = _mlp(features.astype(jnp.float32), W0, b0r, W1p, b1p,
                              psrc, pdst)
    padn = ((0, _NP - _N), (0, 0))
    g = jnp.pad(g, padn)
    bb = jnp.pad(bb, padn)
    g0t = jnp.pad(g0t, padn)
    p = _fused(g, bb, g0t, src, dst, zeros)
    return _final(p, af, c0)
